# baseline (speedup 1.0000x reference)
import functools

import jax
import jax.numpy as jnp
from jax import lax
from jax.experimental import pallas as pl
from jax.experimental.pallas import tpu as pltpu
from jax.experimental.pallas import tpu_sc as plsc

VOCAB = 2048
HIDDEN = 2048
BATCH = 4
SEQ = 2048
NTOK = BATCH * SEQ


def _table_body(e_ref, w_ref, b_ref, t_ref):
    t_ref[...] = lax.dot_general(
        e_ref[...].astype(jnp.bfloat16), w_ref[...].astype(jnp.bfloat16),
        dimension_numbers=(((1,), (1,)), ((), ())),
        preferred_element_type=jnp.float32,
    ) + b_ref[...]


def _build_table(emb_weight, lin_weight, lin_bias):
    BV = 512
    grid = (VOCAB // BV,)
    return pl.pallas_call(
        _table_body,
        grid=grid,
        in_specs=[
            pl.BlockSpec((BV, VOCAB), lambda i: (i, 0)),
            pl.BlockSpec((HIDDEN, VOCAB), lambda i: (0, 0)),
            pl.BlockSpec((1, HIDDEN), lambda i: (0, 0)),
        ],
        out_specs=pl.BlockSpec((BV, HIDDEN), lambda i: (i, 0)),
        out_shape=jax.ShapeDtypeStruct((VOCAB, HIDDEN), jnp.float32),
    )(emb_weight, lin_weight, lin_bias.reshape(1, HIDDEN))


def _make_gather():
    info = plsc.get_sparse_core_info()
    nc, ns = info.num_cores, info.num_subcores
    nw = nc * ns
    b_per_w = NTOK // nw
    chunk = 16
    nbuf = 3
    nchunk = b_per_w // chunk
    mesh = plsc.VectorSubcoreMesh(core_axis_name="c", subcore_axis_name="s")

    @functools.partial(
        pl.kernel, mesh=mesh,
        out_type=jax.ShapeDtypeStruct((NTOK, HIDDEN), jnp.float32),
        scratch_types=[
            pltpu.VMEM((b_per_w,), jnp.int32),
            [pltpu.VMEM((chunk, HIDDEN), jnp.float32) for _ in range(nbuf)],
            [pltpu.SemaphoreType.DMA for _ in range(nbuf)],
            [pltpu.SemaphoreType.DMA for _ in range(nbuf)],
        ],
    )
    def gather(table_hbm, idx_hbm, out_hbm, idx_v, bufs, isems, osems):
        wid = lax.axis_index("s") * nc + lax.axis_index("c")
        base = wid * b_per_w
        pltpu.sync_copy(idx_hbm.at[pl.ds(base, b_per_w)], idx_v)
        g = [None] * nbuf
        o = [None] * nbuf
        g[0] = pltpu.async_copy(
            table_hbm.at[idx_v.at[pl.ds(0, chunk)]], bufs[0], isems[0])
        for c in range(nchunk):
            cur = c % nbuf
            if c + 1 < nchunk:
                nxt = (c + 1) % nbuf
                if o[nxt] is not None:
                    o[nxt].wait()
                g[nxt] = pltpu.async_copy(
                    table_hbm.at[idx_v.at[pl.ds((c + 1) * chunk, chunk)]],
                    bufs[nxt], isems[nxt])
            g[cur].wait()
            o[cur] = pltpu.async_copy(
                bufs[cur], out_hbm.at[pl.ds(base + c * chunk, chunk)],
                osems[cur])
        for j in range(nbuf):
            if o[j] is not None:
                o[j].wait()

    return gather


_gather = _make_gather()


def kernel(x, emb_weight, lin_weight, lin_bias):
    table = _build_table(emb_weight, lin_weight, lin_bias)
    idx = x.reshape(-1).astype(jnp.int32)
    out = _gather(table, idx)
    return out.reshape(BATCH, SEQ, HIDDEN)


# speedup vs baseline: 1.0082x; 1.0082x over previous
import functools

import jax
import jax.numpy as jnp
from jax import lax
from jax.experimental import pallas as pl
from jax.experimental.pallas import tpu as pltpu
from jax.experimental.pallas import tpu_sc as plsc

VOCAB = 2048
HIDDEN = 2048
BATCH = 4
SEQ = 2048
NTOK = BATCH * SEQ


def _table_body(e_ref, w_ref, b_ref, t_ref):
    t_ref[...] = lax.dot_general(
        e_ref[...].astype(jnp.bfloat16), w_ref[...].astype(jnp.bfloat16),
        dimension_numbers=(((1,), (1,)), ((), ())),
        preferred_element_type=jnp.float32,
    ) + b_ref[...]


def _build_table(emb_weight, lin_weight, lin_bias):
    BV = 512
    grid = (VOCAB // BV,)
    return pl.pallas_call(
        _table_body,
        grid=grid,
        in_specs=[
            pl.BlockSpec((BV, VOCAB), lambda i: (i, 0)),
            pl.BlockSpec((HIDDEN, VOCAB), lambda i: (0, 0)),
            pl.BlockSpec((1, HIDDEN), lambda i: (0, 0)),
        ],
        out_specs=pl.BlockSpec((BV, HIDDEN), lambda i: (i, 0)),
        out_shape=jax.ShapeDtypeStruct((VOCAB, HIDDEN), jnp.float32),
    )(emb_weight, lin_weight, lin_bias.reshape(1, HIDDEN))


def _make_gather():
    info = plsc.get_sparse_core_info()
    nc, ns = info.num_cores, info.num_subcores
    nw = nc * ns
    b_per_w = NTOK // nw
    chunk = 24
    sizes = [chunk] * (b_per_w // chunk) + (
        [b_per_w % chunk] if b_per_w % chunk else [])
    starts = [sum(sizes[:j]) for j in range(len(sizes))]
    nchunk = len(sizes)
    mesh = plsc.VectorSubcoreMesh(core_axis_name="c", subcore_axis_name="s")

    @functools.partial(
        pl.kernel, mesh=mesh,
        out_type=jax.ShapeDtypeStruct((NTOK, HIDDEN), jnp.float32),
        scratch_types=[
            pltpu.VMEM((b_per_w,), jnp.int32),
            [pltpu.VMEM((chunk, HIDDEN), jnp.float32) for _ in range(2)],
            [pltpu.SemaphoreType.DMA for _ in range(2)],
        ],
    )
    def gather(table_hbm, idx_hbm, out_hbm, idx_v, bufs, sems):
        wid = lax.axis_index("s") * nc + lax.axis_index("c")
        base = wid * b_per_w
        pltpu.sync_copy(idx_hbm.at[pl.ds(base, b_per_w)], idx_v)
        g = [None, None]
        g[0] = pltpu.async_copy(
            table_hbm.at[idx_v.at[pl.ds(0, sizes[0])]],
            bufs[0].at[pl.ds(0, sizes[0])], sems[0])
        for c in range(nchunk):
            cur = c % 2
            if c + 1 < nchunk:
                nxt = (c + 1) % 2
                g[nxt] = pltpu.async_copy(
                    table_hbm.at[idx_v.at[pl.ds(starts[c + 1], sizes[c + 1])]],
                    bufs[nxt].at[pl.ds(0, sizes[c + 1])], sems[nxt])
            g[cur].wait()
            pltpu.sync_copy(
                bufs[cur].at[pl.ds(0, sizes[c])],
                out_hbm.at[pl.ds(base + starts[c], sizes[c])])

    return gather


_gather = _make_gather()


def kernel(x, emb_weight, lin_weight, lin_bias):
    table = _build_table(emb_weight, lin_weight, lin_bias)
    idx = x.reshape(-1).astype(jnp.int32)
    out = _gather(table, idx)
    return out.reshape(BATCH, SEQ, HIDDEN)


# speedup vs baseline: 1.0082x; 1.0000x over previous
import functools

import jax
import jax.numpy as jnp
from jax import lax
from jax.experimental import pallas as pl
from jax.experimental.pallas import tpu as pltpu
from jax.experimental.pallas import tpu_sc as plsc

VOCAB = 2048
HIDDEN = 2048
BATCH = 4
SEQ = 2048
NTOK = BATCH * SEQ


def _table_body(e_ref, w_ref, b_ref, t_ref):
    t_ref[...] = lax.dot_general(
        e_ref[...].astype(jnp.bfloat16), w_ref[...].astype(jnp.bfloat16),
        dimension_numbers=(((1,), (1,)), ((), ())),
        preferred_element_type=jnp.float32,
    ) + b_ref[...]


def _build_table(emb_weight, lin_weight, lin_bias):
    BV = 256
    grid = (VOCAB // BV,)
    return pl.pallas_call(
        _table_body,
        grid=grid,
        in_specs=[
            pl.BlockSpec((BV, VOCAB), lambda i: (i, 0)),
            pl.BlockSpec((HIDDEN, VOCAB), lambda i: (0, 0)),
            pl.BlockSpec((1, HIDDEN), lambda i: (0, 0)),
        ],
        out_specs=pl.BlockSpec((BV, HIDDEN), lambda i: (i, 0)),
        out_shape=jax.ShapeDtypeStruct((VOCAB, HIDDEN), jnp.float32),
    )(emb_weight, lin_weight, lin_bias.reshape(1, HIDDEN))


def _make_gather():
    info = plsc.get_sparse_core_info()
    nc, ns = info.num_cores, info.num_subcores
    nw = nc * ns
    b_per_w = NTOK // nw
    chunk = 16
    nbuf = 3
    nchunk = b_per_w // chunk
    mesh = plsc.VectorSubcoreMesh(core_axis_name="c", subcore_axis_name="s")

    @functools.partial(
        pl.kernel, mesh=mesh,
        out_type=jax.ShapeDtypeStruct((NTOK, HIDDEN), jnp.float32),
        scratch_types=[
            pltpu.VMEM((b_per_w,), jnp.int32),
            [pltpu.VMEM((chunk, HIDDEN), jnp.float32) for _ in range(nbuf)],
            [pltpu.SemaphoreType.DMA for _ in range(nbuf)],
        ],
    )
    def gather(table_hbm, idx_hbm, out_hbm, idx_v, bufs, sems):
        wid = lax.axis_index("s") * nc + lax.axis_index("c")
        base = wid * b_per_w
        pltpu.sync_copy(idx_hbm.at[pl.ds(base, b_per_w)], idx_v)
        g = [None] * nbuf
        for j in range(nbuf - 1):
            g[j] = pltpu.async_copy(
                table_hbm.at[idx_v.at[pl.ds(j * chunk, chunk)]],
                bufs[j], sems[j])
        for c in range(nchunk):
            cur = c % nbuf
            if c + nbuf - 1 < nchunk:
                nxt = (c + nbuf - 1) % nbuf
                g[nxt] = pltpu.async_copy(
                    table_hbm.at[idx_v.at[pl.ds((c + nbuf - 1) * chunk, chunk)]],
                    bufs[nxt], sems[nxt])
            g[cur].wait()
            pltpu.sync_copy(bufs[cur], out_hbm.at[pl.ds(base + c * chunk, chunk)])

    return gather


_gather = _make_gather()


def kernel(x, emb_weight, lin_weight, lin_bias):
    table = _build_table(emb_weight, lin_weight, lin_bias)
    idx = x.reshape(-1).astype(jnp.int32)
    out = _gather(table, idx)
    return out.reshape(BATCH, SEQ, HIDDEN)


# speedup vs baseline: 1.0244x; 1.0160x over previous
import functools

import jax
import jax.numpy as jnp
from jax import lax
from jax.experimental import pallas as pl
from jax.experimental.pallas import tpu as pltpu
from jax.experimental.pallas import tpu_sc as plsc

VOCAB = 2048
HIDDEN = 2048
BATCH = 4
SEQ = 2048
NTOK = BATCH * SEQ


def _table_body(e_ref, w_ref, b_ref, t_ref):
    t_ref[...] = lax.dot_general(
        e_ref[...].astype(jnp.bfloat16), w_ref[...].astype(jnp.bfloat16),
        dimension_numbers=(((1,), (1,)), ((), ())),
        preferred_element_type=jnp.float32,
    ) + b_ref[...]


def _build_table(emb_weight, lin_weight, lin_bias):
    BV = 512
    grid = (VOCAB // BV,)
    return pl.pallas_call(
        _table_body,
        grid=grid,
        in_specs=[
            pl.BlockSpec((BV, VOCAB), lambda i: (i, 0)),
            pl.BlockSpec((HIDDEN, VOCAB), lambda i: (0, 0)),
            pl.BlockSpec((1, HIDDEN), lambda i: (0, 0)),
        ],
        out_specs=pl.BlockSpec((BV, HIDDEN), lambda i: (i, 0)),
        out_shape=jax.ShapeDtypeStruct((VOCAB, HIDDEN), jnp.float32),
    )(emb_weight, lin_weight, lin_bias.reshape(1, HIDDEN))


def _make_gather():
    info = plsc.get_sparse_core_info()
    nc, ns = info.num_cores, info.num_subcores
    nw = nc * ns
    b_per_w = NTOK // nw
    chunk = 16
    nbuf = 3
    nchunk = b_per_w // chunk
    mesh = plsc.VectorSubcoreMesh(core_axis_name="c", subcore_axis_name="s")
    w_per_row = SEQ // b_per_w

    @functools.partial(
        pl.kernel, mesh=mesh,
        out_type=jax.ShapeDtypeStruct((NTOK, HIDDEN), jnp.float32),
        scratch_types=[
            pltpu.VMEM((b_per_w,), jnp.int32),
            [pltpu.VMEM((chunk, HIDDEN), jnp.float32) for _ in range(nbuf)],
            [pltpu.SemaphoreType.DMA for _ in range(nbuf)],
        ],
    )
    def gather(table_hbm, idx_hbm, out_hbm, idx_v, bufs, sems):
        wid = lax.axis_index("s") * nc + lax.axis_index("c")
        base = wid * b_per_w
        pltpu.sync_copy(
            idx_hbm.at[wid // w_per_row,
                       pl.ds((wid % w_per_row) * b_per_w, b_per_w)],
            idx_v)
        g = [None] * nbuf
        for j in range(nbuf - 1):
            g[j] = pltpu.async_copy(
                table_hbm.at[idx_v.at[pl.ds(j * chunk, chunk)]],
                bufs[j], sems[j])
        for c in range(nchunk):
            cur = c % nbuf
            if c + nbuf - 1 < nchunk:
                nxt = (c + nbuf - 1) % nbuf
                g[nxt] = pltpu.async_copy(
                    table_hbm.at[idx_v.at[pl.ds((c + nbuf - 1) * chunk, chunk)]],
                    bufs[nxt], sems[nxt])
            g[cur].wait()
            pltpu.sync_copy(bufs[cur], out_hbm.at[pl.ds(base + c * chunk, chunk)])

    return gather


_gather = _make_gather()


def kernel(x, emb_weight, lin_weight, lin_bias):
    table = _build_table(emb_weight, lin_weight, lin_bias)
    out = _gather(table, x.astype(jnp.int32))
    return out.reshape(BATCH, SEQ, HIDDEN)
